# Initial kernel scaffold; baseline (speedup 1.0000x reference)
import jax
import jax.numpy as jnp
from jax.experimental import pallas as pl


def kernel(position, batch_index, edge_index, bounds, W1, b1, W2, b2, W3, b3, ln_g, ln_b, We1, be1, We2, be2, We3, be3, eln_g, eln_b):
    raise NotImplementedError("write your pallas kernel here")


# speedup vs baseline: 8.4095x; 8.4095x over previous
import functools

import jax
import jax.numpy as jnp
from jax import lax
from jax.experimental import pallas as pl
from jax.experimental.pallas import tpu as pltpu
from jax.experimental.pallas import tpu_sc as plsc

N = 10000
E = 320000
VSTD = 0.0016
TBL_W = 128
GCHUNK = 80
NODE_BLK = 2000
EDGE_BLK = 4000

_HI = jax.lax.Precision.HIGHEST


def _table_kernel(pos_ref, tbl_ref):
    p = pos_ref[...]
    lp = p[:, 10:12]
    ls = (p[:, 10:12] - p[:, 8:10]) * (1.0 / VSTD)
    z = jnp.zeros((p.shape[0], TBL_W - 4), jnp.float32)
    tbl_ref[...] = jnp.concatenate([lp, ls, z], axis=1)


def _node_kernel(pos_ref, bounds_ref, w1_ref, b1_ref, w2_ref, b2_ref,
                 w3_ref, b3_ref, g_ref, bb_ref, x_ref):
    p = pos_ref[...]
    ns = (p[:, 2:12] - p[:, 0:10]) * (1.0 / VSTD)
    lp = p[:, 10:12]
    lx = p[:, 10:11]
    ly = p[:, 11:12]
    xbd = jnp.clip(lx - bounds_ref[0:1, :], -1.0, 1.0)
    ybd = jnp.clip(ly - bounds_ref[1:2, :], -1.0, 1.0)
    xf = jnp.concatenate([ns, lp, xbd, ybd], axis=1)
    h = jax.nn.relu(jnp.dot(xf, w1_ref[...], precision=_HI) + b1_ref[...])
    h = jax.nn.relu(jnp.dot(h, w2_ref[...], precision=_HI) + b2_ref[...])
    h = jnp.dot(h, w3_ref[...], precision=_HI) + b3_ref[...]
    m = jnp.mean(h, axis=-1, keepdims=True)
    v = jnp.mean((h - m) ** 2, axis=-1, keepdims=True)
    x_ref[...] = (h - m) / jnp.sqrt(v + 1e-5) * g_ref[...] + bb_ref[...]


def _edge_kernel(s_ref, d_ref, w1_ref, b1_ref, w2_ref, b2_ref,
                 w3_ref, b3_ref, g_ref, bb_ref, ea_ref):
    diff = s_ref[...] - d_ref[...]
    dpx = diff[:, 0:1]
    dpy = diff[:, 1:2]
    dsx = diff[:, 2:3]
    dsy = diff[:, 3:4]
    dist = jnp.sqrt(dpx * dpx + dpy * dpy)
    absv = jnp.sqrt(dsx * dsx + dsy * dsy)
    rm = (dsx * dpx + dsy * dpy) / (absv + 1e-6) / (dist + 1e-6)
    z = jnp.zeros_like(dpx)
    feat = jnp.concatenate([dpx, dpy, dist, dsx, dsy, absv, rm, z], axis=1)
    h = jax.nn.relu(jnp.dot(feat, w1_ref[...], precision=_HI) + b1_ref[...])
    h = jax.nn.relu(jnp.dot(h, w2_ref[...], precision=_HI) + b2_ref[...])
    h = jnp.dot(h, w3_ref[...], precision=_HI) + b3_ref[...]
    m = jnp.mean(h, axis=-1, keepdims=True)
    v = jnp.mean((h - m) ** 2, axis=-1, keepdims=True)
    ea_ref[...] = (h - m) / jnp.sqrt(v + 1e-5) * g_ref[...] + bb_ref[...]


def _sc_gather(table, idx):
    info = plsc.get_sparse_core_info()
    nc, ns = info.num_cores, info.num_subcores
    nw = nc * ns
    b = idx.shape[0]
    b_per_w = b // nw
    mesh = plsc.VectorSubcoreMesh(core_axis_name="c", subcore_axis_name="s")

    @functools.partial(
        pl.kernel, mesh=mesh,
        out_type=jax.ShapeDtypeStruct((b, TBL_W), jnp.float32),
        scratch_types=[
            pltpu.VMEM((GCHUNK,), jnp.int32),
            pltpu.VMEM((GCHUNK, TBL_W), jnp.float32),
            pltpu.SemaphoreType.DMA,
        ],
    )
    def k(table_hbm, idx_hbm, out_hbm, idx_v, rows_v, sem):
        wid = lax.axis_index("s") * nc + lax.axis_index("c")
        base = wid * b_per_w

        @pl.loop(0, b_per_w, step=GCHUNK)
        def _(off):
            pltpu.sync_copy(idx_hbm.at[pl.ds(base + off, GCHUNK)], idx_v)
            pltpu.async_copy(table_hbm.at[idx_v], rows_v, sem).wait()
            pltpu.sync_copy(rows_v, out_hbm.at[pl.ds(base + off, GCHUNK)])

    return k(table, idx)


def kernel(position, batch_index, edge_index, bounds, W1, b1, W2, b2, W3, b3,
           ln_g, ln_b, We1, be1, We2, be2, We3, be3, eln_g, eln_b):
    p2d = position.reshape(N, 12)

    table = pl.pallas_call(
        _table_kernel,
        grid=(N // NODE_BLK,),
        in_specs=[pl.BlockSpec((NODE_BLK, 12), lambda i: (i, 0))],
        out_specs=pl.BlockSpec((NODE_BLK, TBL_W), lambda i: (i, 0)),
        out_shape=jax.ShapeDtypeStruct((N, TBL_W), jnp.float32),
    )(p2d)

    idx = edge_index.reshape(2 * E).astype(jnp.int32)
    gathered = _sc_gather(table, idx)

    full = lambda shape: pl.BlockSpec(shape, lambda i: (0,) * len(shape))
    x = pl.pallas_call(
        _node_kernel,
        grid=(N // NODE_BLK,),
        in_specs=[
            pl.BlockSpec((NODE_BLK, 12), lambda i: (i, 0)),
            full((2, 2)),
            full((16, 32)), full((1, 32)),
            full((32, 64)), full((1, 64)),
            full((64, 128)), full((1, 128)),
            full((1, 128)), full((1, 128)),
        ],
        out_specs=pl.BlockSpec((NODE_BLK, 128), lambda i: (i, 0)),
        out_shape=jax.ShapeDtypeStruct((N, 128), jnp.float32),
    )(p2d, bounds, W1.T, b1.reshape(1, 32), W2.T, b2.reshape(1, 64),
      W3.T, b3.reshape(1, 128), ln_g.reshape(1, 128), ln_b.reshape(1, 128))

    We1p = jnp.pad(We1, ((0, 0), (0, 1))).T
    nblk = E // EDGE_BLK
    ea = pl.pallas_call(
        _edge_kernel,
        grid=(nblk,),
        in_specs=[
            pl.BlockSpec((EDGE_BLK, TBL_W), lambda i: (i, 0)),
            pl.BlockSpec((EDGE_BLK, TBL_W), lambda i: (i + nblk, 0)),
            full((8, 32)), full((1, 32)),
            full((32, 64)), full((1, 64)),
            full((64, 128)), full((1, 128)),
            full((1, 128)), full((1, 128)),
        ],
        out_specs=pl.BlockSpec((EDGE_BLK, 128), lambda i: (i, 0)),
        out_shape=jax.ShapeDtypeStruct((E, 128), jnp.float32),
    )(gathered, gathered, We1p, be1.reshape(1, 32), We2.T, be2.reshape(1, 64),
      We3.T, be3.reshape(1, 128), eln_g.reshape(1, 128), eln_b.reshape(1, 128))

    return (x, edge_index, ea, batch_index)


# speedup vs baseline: 11.5720x; 1.3761x over previous
import dataclasses
import functools

import jax
import jax.numpy as jnp
from jax import lax
from jax.experimental import pallas as pl
from jax.experimental.pallas import tpu as pltpu
from jax.experimental.pallas import tpu_sc as plsc

N = 10000
E = 320000
VSTD = 0.0016
GCHUNK = 4000
NQ = 2 * E // GCHUNK
NODE_BLK = 2000

_HI = jax.lax.Precision.HIGHEST


def _table_kernel(pos_ref, tbl_ref):
    p = pos_ref[...]
    lp = p[:, 10:12]
    ls = (p[:, 10:12] - p[:, 8:10]) * (1.0 / VSTD)
    tbl_ref[...] = jnp.concatenate([lp, ls], axis=1)


def _node_kernel(pos_ref, bounds_ref, w1_ref, b1_ref, w2_ref, b2_ref,
                 w3_ref, b3_ref, g_ref, bb_ref, x_ref):
    p = pos_ref[...]
    ns = (p[:, 2:12] - p[:, 0:10]) * (1.0 / VSTD)
    lp = p[:, 10:12]
    lx = p[:, 10:11]
    ly = p[:, 11:12]
    xbd = jnp.clip(lx - bounds_ref[0:1, :], -1.0, 1.0)
    ybd = jnp.clip(ly - bounds_ref[1:2, :], -1.0, 1.0)
    xf = jnp.concatenate([ns, lp, xbd, ybd], axis=1)
    h = jax.nn.relu(jnp.dot(xf, w1_ref[...], precision=_HI) + b1_ref[...])
    h = jax.nn.relu(jnp.dot(h, w2_ref[...], precision=_HI) + b2_ref[...])
    h = jnp.dot(h, w3_ref[...], precision=_HI) + b3_ref[...]
    m = jnp.mean(h, axis=-1, keepdims=True)
    v = jnp.mean((h - m) ** 2, axis=-1, keepdims=True)
    x_ref[...] = (h - m) / jnp.sqrt(v + 1e-5) * g_ref[...] + bb_ref[...]


def _edge_kernel(s_ref, d_ref, w1_ref, b1_ref, w2_ref, b2_ref,
                 w3_ref, b3_ref, gg_ref, bb_ref, ea_ref):
    s = s_ref[...].reshape(4, GCHUNK)
    d = d_ref[...].reshape(4, GCHUNK)
    diff = jnp.transpose(s - d)
    dpx = diff[:, 0:1]
    dpy = diff[:, 1:2]
    dsx = diff[:, 2:3]
    dsy = diff[:, 3:4]
    dist = jnp.sqrt(dpx * dpx + dpy * dpy)
    absv = jnp.sqrt(dsx * dsx + dsy * dsy)
    rm = (dsx * dpx + dsy * dpy) / (absv + 1e-6) / (dist + 1e-6)
    z = jnp.zeros_like(dpx)
    feat = jnp.concatenate([dpx, dpy, dist, dsx, dsy, absv, rm, z], axis=1)
    h = jax.nn.relu(jnp.dot(feat, w1_ref[...], precision=_HI) + b1_ref[...])
    h = jax.nn.relu(jnp.dot(h, w2_ref[...], precision=_HI) + b2_ref[...])
    h = jnp.dot(h, w3_ref[...], precision=_HI) + b3_ref[...]
    m = jnp.mean(h, axis=-1, keepdims=True)
    v = jnp.mean((h - m) ** 2, axis=-1, keepdims=True)
    ea_ref[...] = (h - m) / jnp.sqrt(v + 1e-5) * gg_ref[...] + bb_ref[...]


def _sc_gather(table_flat, idx):
    info = plsc.get_sparse_core_info()
    nc = info.num_cores
    nw = nc * info.num_subcores
    q_per_w = NQ // nw
    mesh = plsc.VectorSubcoreMesh(core_axis_name="c", subcore_axis_name="s")
    cp = pltpu.CompilerParams()
    if "needs_layout_passes" in pltpu.CompilerParams.__dataclass_fields__:
        cp = dataclasses.replace(cp, needs_layout_passes=False)

    @functools.partial(
        pl.kernel, mesh=mesh, compiler_params=cp,
        out_type=jax.ShapeDtypeStruct((NQ, 4, GCHUNK), jnp.float32),
        scratch_types=[
            pltpu.VMEM((table_flat.shape[0],), jnp.float32),
            pltpu.VMEM((GCHUNK,), jnp.int32),
            pltpu.VMEM((4, GCHUNK), jnp.float32),
        ],
    )
    def k(tbl_hbm, idx_hbm, out_hbm, tbl_v, idx_v, out_v):
        wid = lax.axis_index("s") * nc + lax.axis_index("c")
        pltpu.sync_copy(tbl_hbm, tbl_v)

        @pl.loop(0, q_per_w)
        def _(q):
            t = wid * q_per_w + q
            pltpu.sync_copy(idx_hbm.at[pl.ds(t * GCHUNK, GCHUNK)], idx_v)

            @pl.loop(0, GCHUNK, step=16)
            def _(j):
                g = idx_v[pl.ds(j, 16)] * 4
                for c in range(4):
                    out_v[c, pl.ds(j, 16)] = plsc.load_gather(tbl_v, [g + c])

            pltpu.sync_copy(out_v, out_hbm.at[t])

    return k(table_flat, idx)


def kernel(position, batch_index, edge_index, bounds, W1, b1, W2, b2, W3, b3,
           ln_g, ln_b, We1, be1, We2, be2, We3, be3, eln_g, eln_b):
    p2d = position.reshape(N, 12)

    table = pl.pallas_call(
        _table_kernel,
        grid=(N // NODE_BLK,),
        in_specs=[pl.BlockSpec((NODE_BLK, 12), lambda i: (i, 0))],
        out_specs=pl.BlockSpec((NODE_BLK, 4), lambda i: (i, 0)),
        out_shape=jax.ShapeDtypeStruct((N, 4), jnp.float32),
    )(p2d)

    idx = edge_index.reshape(2 * E).astype(jnp.int32)
    gathered = _sc_gather(table.reshape(4 * N), idx)

    full = lambda shape: pl.BlockSpec(shape, lambda i: (0,) * len(shape))
    x = pl.pallas_call(
        _node_kernel,
        grid=(N // NODE_BLK,),
        in_specs=[
            pl.BlockSpec((NODE_BLK, 12), lambda i: (i, 0)),
            full((2, 2)),
            full((16, 32)), full((1, 32)),
            full((32, 64)), full((1, 64)),
            full((64, 128)), full((1, 128)),
            full((1, 128)), full((1, 128)),
        ],
        out_specs=pl.BlockSpec((NODE_BLK, 128), lambda i: (i, 0)),
        out_shape=jax.ShapeDtypeStruct((N, 128), jnp.float32),
    )(p2d, bounds, W1.T, b1.reshape(1, 32), W2.T, b2.reshape(1, 64),
      W3.T, b3.reshape(1, 128), ln_g.reshape(1, 128), ln_b.reshape(1, 128))

    We1p = jnp.pad(We1, ((0, 0), (0, 1))).T
    nsrc = NQ // 2
    ea = pl.pallas_call(
        _edge_kernel,
        grid=(nsrc,),
        in_specs=[
            pl.BlockSpec((1, 4, GCHUNK), lambda i: (i, 0, 0)),
            pl.BlockSpec((1, 4, GCHUNK), lambda i: (i + nsrc, 0, 0)),
            full((8, 32)), full((1, 32)),
            full((32, 64)), full((1, 64)),
            full((64, 128)), full((1, 128)),
            full((1, 128)), full((1, 128)),
        ],
        out_specs=pl.BlockSpec((GCHUNK, 128), lambda i: (i, 0)),
        out_shape=jax.ShapeDtypeStruct((E, 128), jnp.float32),
    )(gathered, gathered, We1p, be1.reshape(1, 32), We2.T, be2.reshape(1, 64),
      We3.T, be3.reshape(1, 128), eln_g.reshape(1, 128), eln_b.reshape(1, 128))

    return (x, edge_index, ea, batch_index)


# speedup vs baseline: 33.6683x; 2.9095x over previous
import dataclasses
import functools

import jax
import jax.numpy as jnp
from jax import lax
from jax.experimental import pallas as pl
from jax.experimental.pallas import tpu as pltpu
from jax.experimental.pallas import tpu_sc as plsc

N = 10000
E = 320000
VSTD = 0.0016
GCHUNK = 2000
NQ = E // GCHUNK
NODE_BLK = 2000

_HI = jax.lax.Precision.HIGHEST


def _table_kernel(pos_ref, tbl_ref):
    p = pos_ref[...]
    lp = p[:, 10:12]
    ls = (p[:, 10:12] - p[:, 8:10]) * (1.0 / VSTD)
    tbl_ref[...] = jnp.concatenate([lp, ls], axis=1)


def _node_kernel(pos_ref, bounds_ref, w1_ref, b1_ref, w2_ref, b2_ref,
                 w3_ref, b3_ref, g_ref, bb_ref, x_ref):
    p = pos_ref[...]
    ns = (p[:, 2:12] - p[:, 0:10]) * (1.0 / VSTD)
    lp = p[:, 10:12]
    lx = p[:, 10:11]
    ly = p[:, 11:12]
    xbd = jnp.clip(lx - bounds_ref[0:1, :], -1.0, 1.0)
    ybd = jnp.clip(ly - bounds_ref[1:2, :], -1.0, 1.0)
    xf = jnp.concatenate([ns, lp, xbd, ybd], axis=1)
    h = jax.nn.relu(jnp.dot(xf, w1_ref[...], precision=_HI) + b1_ref[...])
    h = jax.nn.relu(jnp.dot(h, w2_ref[...], precision=_HI) + b2_ref[...])
    h = jnp.dot(h, w3_ref[...], precision=_HI) + b3_ref[...]
    m = jnp.mean(h, axis=-1, keepdims=True)
    v = jnp.mean((h - m) ** 2, axis=-1, keepdims=True)
    x_ref[...] = (h - m) / jnp.sqrt(v + 1e-5) * g_ref[...] + bb_ref[...]


def _edge_kernel(g_ref, w1_ref, b1_ref, w2_ref, b2_ref,
                 w3_ref, b3_ref, gg_ref, bb_ref, ea_ref):
    m = g_ref[...].reshape(8, GCHUNK)
    diff = m[0:4] - m[4:8]
    dpx = diff[0:1]
    dpy = diff[1:2]
    dsx = diff[2:3]
    dsy = diff[3:4]
    dist = jnp.sqrt(dpx * dpx + dpy * dpy)
    absv = jnp.sqrt(dsx * dsx + dsy * dsy)
    rm = (dsx * dpx + dsy * dpy) / (absv + 1e-6) / (dist + 1e-6)
    feat = jnp.concatenate(
        [diff[0:2], dist, diff[2:4], absv, rm, jnp.zeros_like(rm)], axis=0)
    h = jax.nn.relu(jnp.dot(w1_ref[...], feat, precision=_HI) + b1_ref[...])
    h = jax.nn.relu(jnp.dot(w2_ref[...], h, precision=_HI) + b2_ref[...])
    h = jnp.dot(w3_ref[...], h, precision=_HI) + b3_ref[...]
    mn = jnp.mean(h, axis=0, keepdims=True)
    hc = h - mn
    v = jnp.mean(hc * hc, axis=0, keepdims=True)
    out = hc * jax.lax.rsqrt(v + 1e-5) * gg_ref[...] + bb_ref[...]
    ea_ref[...] = jnp.transpose(out)


def _sc_gather(table_flat, idx):
    info = plsc.get_sparse_core_info()
    nc = info.num_cores
    nw = nc * info.num_subcores
    q_per_w = NQ // nw
    mesh = plsc.VectorSubcoreMesh(core_axis_name="c", subcore_axis_name="s")
    cp = pltpu.CompilerParams()
    if "needs_layout_passes" in pltpu.CompilerParams.__dataclass_fields__:
        cp = dataclasses.replace(cp, needs_layout_passes=False)

    @functools.partial(
        pl.kernel, mesh=mesh, compiler_params=cp,
        out_type=jax.ShapeDtypeStruct((NQ, 8, GCHUNK), jnp.float32),
        scratch_types=[
            pltpu.VMEM((table_flat.shape[0],), jnp.float32),
            pltpu.VMEM((GCHUNK,), jnp.int32),
            pltpu.VMEM((8, GCHUNK), jnp.float32),
        ],
    )
    def k(tbl_hbm, idx_hbm, out_hbm, tbl_v, idx_v, out_v):
        wid = lax.axis_index("s") * nc + lax.axis_index("c")
        pltpu.sync_copy(tbl_hbm, tbl_v)

        @pl.loop(0, q_per_w)
        def _(q):
            t = wid * q_per_w + q
            for half in range(2):
                pltpu.sync_copy(
                    idx_hbm.at[pl.ds(half * E + t * GCHUNK, GCHUNK)], idx_v)

                @pl.loop(0, GCHUNK, step=16)
                def _(j):
                    g = idx_v[pl.ds(j, 16)] * 4
                    for c in range(4):
                        out_v[4 * half + c, pl.ds(j, 16)] = (
                            plsc.load_gather(tbl_v, [g + c]))

            pltpu.sync_copy(out_v, out_hbm.at[t])

    return k(table_flat, idx)


def kernel(position, batch_index, edge_index, bounds, W1, b1, W2, b2, W3, b3,
           ln_g, ln_b, We1, be1, We2, be2, We3, be3, eln_g, eln_b):
    p2d = position.reshape(N, 12)

    table = pl.pallas_call(
        _table_kernel,
        grid=(N // NODE_BLK,),
        in_specs=[pl.BlockSpec((NODE_BLK, 12), lambda i: (i, 0))],
        out_specs=pl.BlockSpec((NODE_BLK, 4), lambda i: (i, 0)),
        out_shape=jax.ShapeDtypeStruct((N, 4), jnp.float32),
    )(p2d)

    idx = edge_index.reshape(2 * E).astype(jnp.int32)
    gathered = _sc_gather(table.reshape(4 * N), idx)

    full = lambda shape: pl.BlockSpec(shape, lambda i: (0,) * len(shape))
    x = pl.pallas_call(
        _node_kernel,
        grid=(N // NODE_BLK,),
        in_specs=[
            pl.BlockSpec((NODE_BLK, 12), lambda i: (i, 0)),
            full((2, 2)),
            full((16, 32)), full((1, 32)),
            full((32, 64)), full((1, 64)),
            full((64, 128)), full((1, 128)),
            full((1, 128)), full((1, 128)),
        ],
        out_specs=pl.BlockSpec((NODE_BLK, 128), lambda i: (i, 0)),
        out_shape=jax.ShapeDtypeStruct((N, 128), jnp.float32),
    )(p2d, bounds, W1.T, b1.reshape(1, 32), W2.T, b2.reshape(1, 64),
      W3.T, b3.reshape(1, 128), ln_g.reshape(1, 128), ln_b.reshape(1, 128))

    We1p = jnp.pad(We1, ((0, 0), (0, 1)))
    ea = pl.pallas_call(
        _edge_kernel,
        grid=(NQ,),
        in_specs=[
            pl.BlockSpec((1, 8, GCHUNK), lambda i: (i, 0, 0)),
            full((32, 8)), full((32, 1)),
            full((64, 32)), full((64, 1)),
            full((128, 64)), full((128, 1)),
            full((128, 1)), full((128, 1)),
        ],
        out_specs=pl.BlockSpec((GCHUNK, 128), lambda i: (i, 0)),
        out_shape=jax.ShapeDtypeStruct((E, 128), jnp.float32),
    )(gathered, We1p, be1.reshape(32, 1), We2, be2.reshape(64, 1),
      We3, be3.reshape(128, 1), eln_g.reshape(128, 1), eln_b.reshape(128, 1))

    return (x, edge_index, ea, batch_index)


# speedup vs baseline: 60.8538x; 1.8074x over previous
import dataclasses
import functools

import jax
import jax.numpy as jnp
from jax import lax
from jax.experimental import pallas as pl
from jax.experimental.pallas import tpu as pltpu
from jax.experimental.pallas import tpu_sc as plsc

N = 10000
E = 320000
VSTD = 0.0016
GCHUNK = 4000
NQ = E // GCHUNK
NODE_BLK = 2000

_HI = jax.lax.Precision.DEFAULT


def _table_kernel(pos_ref, tbl_ref):
    p = pos_ref[...]
    lp = p[:, 10:12]
    ls = (p[:, 10:12] - p[:, 8:10]) * (1.0 / VSTD)
    tbl_ref[...] = jnp.concatenate([lp, ls], axis=1)


def _node_kernel(pos_ref, bounds_ref, w1_ref, b1_ref, w2_ref, b2_ref,
                 w3_ref, b3_ref, g_ref, bb_ref, x_ref):
    p = pos_ref[...]
    ns = (p[:, 2:12] - p[:, 0:10]) * (1.0 / VSTD)
    lp = p[:, 10:12]
    lx = p[:, 10:11]
    ly = p[:, 11:12]
    xbd = jnp.clip(lx - bounds_ref[0:1, :], -1.0, 1.0)
    ybd = jnp.clip(ly - bounds_ref[1:2, :], -1.0, 1.0)
    xf = jnp.concatenate([ns, lp, xbd, ybd], axis=1)
    h = jax.nn.relu(jnp.dot(xf, w1_ref[...], precision=_HI) + b1_ref[...])
    h = jax.nn.relu(jnp.dot(h, w2_ref[...], precision=_HI) + b2_ref[...])
    h = jnp.dot(h, w3_ref[...], precision=_HI) + b3_ref[...]
    m = jnp.mean(h, axis=-1, keepdims=True)
    v = jnp.mean((h - m) ** 2, axis=-1, keepdims=True)
    x_ref[...] = (h - m) / jnp.sqrt(v + 1e-5) * g_ref[...] + bb_ref[...]


def _edge_kernel(g_ref, w1_ref, b1_ref, w2_ref, b2_ref,
                 w3_ref, b3_ref, gg_ref, bb_ref, ea_ref):
    m = g_ref[...].reshape(8, GCHUNK)
    diff = m[0:4] - m[4:8]
    dpx = diff[0:1]
    dpy = diff[1:2]
    dsx = diff[2:3]
    dsy = diff[3:4]
    dist = jnp.sqrt(dpx * dpx + dpy * dpy)
    absv = jnp.sqrt(dsx * dsx + dsy * dsy)
    rm = (dsx * dpx + dsy * dpy) / (absv + 1e-6) / (dist + 1e-6)
    feat = jnp.concatenate(
        [diff[0:2], dist, diff[2:4], absv, rm, jnp.zeros_like(rm)], axis=0)
    h = jax.nn.relu(jnp.dot(w1_ref[...], feat, precision=_HI) + b1_ref[...])
    h = jax.nn.relu(jnp.dot(w2_ref[...], h, precision=_HI) + b2_ref[...])
    h = jnp.dot(w3_ref[...], h, precision=_HI) + b3_ref[...]
    mn = jnp.mean(h, axis=0, keepdims=True)
    hc = h - mn
    v = jnp.mean(hc * hc, axis=0, keepdims=True)
    out = hc * jax.lax.rsqrt(v + 1e-5) * gg_ref[...] + bb_ref[...]
    ea_ref[...] = jnp.transpose(out)


def _sc_gather(table_flat, idx):
    info = plsc.get_sparse_core_info()
    nc = info.num_cores
    nw = nc * info.num_subcores
    q_rounds = (NQ + nw - 1) // nw
    mesh = plsc.VectorSubcoreMesh(core_axis_name="c", subcore_axis_name="s")
    cp = pltpu.CompilerParams()
    if "needs_layout_passes" in pltpu.CompilerParams.__dataclass_fields__:
        cp = dataclasses.replace(cp, needs_layout_passes=False)

    @functools.partial(
        pl.kernel, mesh=mesh, compiler_params=cp,
        out_type=jax.ShapeDtypeStruct((NQ, 8, GCHUNK), jnp.float32),
        scratch_types=[
            pltpu.VMEM((table_flat.shape[0],), jnp.float32),
            pltpu.VMEM((GCHUNK,), jnp.int32),
            pltpu.VMEM((8, GCHUNK), jnp.float32),
        ],
    )
    def k(tbl_hbm, idx_hbm, out_hbm, tbl_v, idx_v, out_v):
        wid = lax.axis_index("s") * nc + lax.axis_index("c")
        pltpu.sync_copy(tbl_hbm, tbl_v)

        @pl.loop(0, q_rounds)
        def _(q):
            t = q * nw + wid

            @pl.when(t < NQ)
            def _():
                for half in range(2):
                    pltpu.sync_copy(
                        idx_hbm.at[pl.ds(half * E + t * GCHUNK, GCHUNK)],
                        idx_v)

                    @pl.loop(0, GCHUNK, step=16)
                    def _(j):
                        g = idx_v[pl.ds(j, 16)] * 4
                        for c in range(4):
                            out_v[4 * half + c, pl.ds(j, 16)] = (
                                plsc.load_gather(tbl_v, [g + c]))

                pltpu.sync_copy(out_v, out_hbm.at[t])

    return k(table_flat, idx)


def kernel(position, batch_index, edge_index, bounds, W1, b1, W2, b2, W3, b3,
           ln_g, ln_b, We1, be1, We2, be2, We3, be3, eln_g, eln_b):
    p2d = position.reshape(N, 12)

    table = pl.pallas_call(
        _table_kernel,
        grid=(N // NODE_BLK,),
        in_specs=[pl.BlockSpec((NODE_BLK, 12), lambda i: (i, 0))],
        out_specs=pl.BlockSpec((NODE_BLK, 4), lambda i: (i, 0)),
        out_shape=jax.ShapeDtypeStruct((N, 4), jnp.float32),
    )(p2d)

    idx = edge_index.reshape(2 * E).astype(jnp.int32)
    gathered = _sc_gather(table.reshape(4 * N), idx)

    full = lambda shape: pl.BlockSpec(shape, lambda i: (0,) * len(shape))
    x = pl.pallas_call(
        _node_kernel,
        grid=(N // NODE_BLK,),
        in_specs=[
            pl.BlockSpec((NODE_BLK, 12), lambda i: (i, 0)),
            full((2, 2)),
            full((16, 32)), full((1, 32)),
            full((32, 64)), full((1, 64)),
            full((64, 128)), full((1, 128)),
            full((1, 128)), full((1, 128)),
        ],
        out_specs=pl.BlockSpec((NODE_BLK, 128), lambda i: (i, 0)),
        out_shape=jax.ShapeDtypeStruct((N, 128), jnp.float32),
    )(p2d, bounds, W1.T, b1.reshape(1, 32), W2.T, b2.reshape(1, 64),
      W3.T, b3.reshape(1, 128), ln_g.reshape(1, 128), ln_b.reshape(1, 128))

    We1p = jnp.pad(We1, ((0, 0), (0, 1)))
    ea = pl.pallas_call(
        _edge_kernel,
        grid=(NQ,),
        in_specs=[
            pl.BlockSpec((1, 8, GCHUNK), lambda i: (i, 0, 0)),
            full((32, 8)), full((32, 1)),
            full((64, 32)), full((64, 1)),
            full((128, 64)), full((128, 1)),
            full((128, 1)), full((128, 1)),
        ],
        out_specs=pl.BlockSpec((GCHUNK, 128), lambda i: (i, 0)),
        out_shape=jax.ShapeDtypeStruct((E, 128), jnp.float32),
    )(gathered, We1p, be1.reshape(32, 1), We2, be2.reshape(64, 1),
      We3, be3.reshape(128, 1), eln_g.reshape(128, 1), eln_b.reshape(128, 1))

    return (x, edge_index, ea, batch_index)
